# Initial kernel scaffold; baseline (speedup 1.0000x reference)
import jax
import jax.numpy as jnp
from jax.experimental import pallas as pl


def kernel(indices, table):
    raise NotImplementedError("write your pallas kernel here")


# speedup vs baseline: 13.4840x; 13.4840x over previous
import functools

import jax
import jax.numpy as jnp
from jax import lax
from jax.experimental import pallas as pl
from jax.experimental.pallas import tpu as pltpu
from jax.experimental.pallas import tpu_sc as plsc

VOCAB = 100001
D = 128
B = 4096
L = 200

NUM_CORES = 2
NUM_SUBCORES = 16
NW = NUM_CORES * NUM_SUBCORES
S_PER_W = B // NW
LANES = 16
NJ = D // LANES

SPLIT = 128


def _sc_body(idx_hbm, table_hbm, out_hbm, idx_v, rows0, rows1, out_v, sem0, sem1):
    wid = lax.axis_index("s") * NUM_CORES + lax.axis_index("c")
    base = wid * (S_PER_W * L)

    pltpu.sync_copy(idx_hbm.at[pl.ds(base, S_PER_W * L)], idx_v)

    def start_gather(i, rows, sem):
        off = pl.multiple_of(i * L, 8)
        pltpu.async_copy(
            table_hbm.at[idx_v.at[pl.ds(off, SPLIT)]],
            rows.at[pl.ds(0, SPLIT)], sem)
        pltpu.async_copy(
            table_hbm.at[idx_v.at[pl.ds(off + SPLIT, L - SPLIT)]],
            rows.at[pl.ds(SPLIT, L - SPLIT)], sem)

    def wait_gather(rows, sem):
        pltpu.make_async_copy(table_hbm.at[pl.ds(0, L)], rows, sem).wait()

    def accumulate(rows, s):
        def body(i, acc):
            r = pl.multiple_of(i * 2, 2)
            acc = tuple(acc[j] + rows[r, pl.ds(j * LANES, LANES)]
                        for j in range(NJ))
            return tuple(acc[j] + rows[r + 1, pl.ds(j * LANES, LANES)]
                         for j in range(NJ))
        acc0 = tuple(jnp.zeros((LANES,), jnp.float32) for _ in range(NJ))
        acc = lax.fori_loop(0, L // 2, body, acc0)
        inv = jnp.float32(1.0 / L)
        for j in range(NJ):
            out_v[s, pl.ds(j * LANES, LANES)] = acc[j] * inv

    start_gather(0, rows0, sem0)
    start_gather(1, rows1, sem1)

    def outer(k, carry):
        a = k * 2
        wait_gather(rows0, sem0)
        accumulate(rows0, a)

        @pl.when(a + 2 < S_PER_W)
        def _():
            start_gather(a + 2, rows0, sem0)

        wait_gather(rows1, sem1)
        accumulate(rows1, a + 1)

        @pl.when(a + 3 < S_PER_W)
        def _():
            start_gather(a + 3, rows1, sem1)

        return carry

    lax.fori_loop(0, S_PER_W // 2, outer, 0)

    pltpu.sync_copy(out_v, out_hbm.at[pl.ds(wid * S_PER_W, S_PER_W)])


@functools.partial(
    pl.kernel,
    out_type=jax.ShapeDtypeStruct((B, D), jnp.float32),
    mesh=plsc.VectorSubcoreMesh(core_axis_name="c", subcore_axis_name="s"),
    scratch_types=[
        pltpu.VMEM((S_PER_W * L,), jnp.int32),
        pltpu.VMEM((L, D), jnp.float32),
        pltpu.VMEM((L, D), jnp.float32),
        pltpu.VMEM((S_PER_W, D), jnp.float32),
        pltpu.SemaphoreType.DMA,
        pltpu.SemaphoreType.DMA,
    ],
)
def _sc_avg(idx_hbm, table_hbm, out_hbm, idx_v, rows0, rows1, out_v, sem0, sem1):
    _sc_body(idx_hbm, table_hbm, out_hbm, idx_v, rows0, rows1, out_v, sem0, sem1)


def kernel(indices, table):
    idx_flat = indices.astype(jnp.int32).reshape(B * L)
    return _sc_avg(idx_flat, table)
